# Initial kernel scaffold; baseline (speedup 1.0000x reference)
import jax
import jax.numpy as jnp
from jax.experimental import pallas as pl


def kernel(Tleaf, Vcmax25, Jmax25, TPU25, Rd25, dHa_Vcmax, dHa_Jmax, dHa_TPU, Topt_Vcmax, Topt_Jmax, Topt_TPU, PIDs, lengths):
    raise NotImplementedError("write your pallas kernel here")


# speedup vs baseline: 1204.0822x; 1204.0822x over previous
import functools

import jax
import jax.numpy as jnp
from jax import lax
from jax.experimental import pallas as pl
from jax.experimental.pallas import tpu as pltpu
from jax.experimental.pallas import tpu_sc as plsc

NUM_PIDS = 1024
SEG = 8192
LEN = 128
TOTAL = SEG * LEN

R_GAS = 0.0083144598
KELVIN = 273.15
TROOM = 25.0 + KELVIN
DHA_RD = 46.39
DHD_VCMAX = 200.0
DHD_JMAX = 200.0
DHD_TPU = 201.8

NC = 2
NS = 16
LANES = 16
NW = NC * NS
SEG_PER_W = SEG // NW


def _sc_gather_body(pids_hbm, dV_hbm, dJ_hbm, dT_hbm, tV_hbm, tJ_hbm, tT_hbm,
                    o_a1V, o_a2V, o_tpV, o_a1J, o_a2J, o_tpJ,
                    o_a1T, o_a2T, o_tpT,
                    pids_v, pids8_v, dVv, dJv, dTv, tVv, tJv, tTv,
                    v_a1V, v_a2V, v_tpV, v_a1J, v_a2J, v_tpJ,
                    v_a1T, v_a2T, v_tpT):
    wid = lax.axis_index("s") * NC + lax.axis_index("c")
    base = wid * SEG_PER_W
    pltpu.sync_copy(pids_hbm.at[pl.ds(base, SEG_PER_W)], pids_v)
    pltpu.sync_copy(pids_hbm.at[pl.ds(0, LANES)], pids8_v)
    pltpu.sync_copy(dV_hbm, dVv)
    pltpu.sync_copy(dJ_hbm, dJv)
    pltpu.sync_copy(dT_hbm, dTv)
    pltpu.sync_copy(tV_hbm, tVv)
    pltpu.sync_copy(tJ_hbm, tJv)
    pltpu.sync_copy(tT_hbm, tTv)
    bufs = (v_a1V, v_a2V, v_tpV, v_a1J, v_a2J, v_tpJ, v_a1T, v_a2T, v_tpT)
    for i in range(SEG_PER_W // LANES):
        sl = pl.ds(i * LANES, LANES)
        p = pids_v[sl]
        q = plsc.load_gather(pids8_v, [jnp.right_shift(p, 7)])
        for ch, (dv, tv) in enumerate(((dVv, tVv), (dJv, tJv), (dTv, tTv))):
            bufs[3 * ch + 0][sl] = plsc.load_gather(dv, [p])
            bufs[3 * ch + 1][sl] = plsc.load_gather(dv, [q])
            bufs[3 * ch + 2][sl] = plsc.load_gather(tv, [p])
    for v, o in zip(bufs, (o_a1V, o_a2V, o_tpV, o_a1J, o_a2J, o_tpJ,
                           o_a1T, o_a2T, o_tpT)):
        pltpu.sync_copy(v, o.at[pl.ds(base, SEG_PER_W)])


def _sc_gather(pids, dV, dJ, dT, tV, tJ, tT):
    out_t = tuple(jax.ShapeDtypeStruct((SEG,), jnp.float32) for _ in range(9))
    mesh = plsc.VectorSubcoreMesh(core_axis_name="c", subcore_axis_name="s",
                                  num_cores=NC, num_subcores=NS)
    return pl.kernel(
        _sc_gather_body,
        out_type=out_t,
        mesh=mesh,
        compiler_params=pltpu.CompilerParams(needs_layout_passes=False),
        scratch_types=[
            pltpu.VMEM((SEG_PER_W,), jnp.int32),
            pltpu.VMEM((LANES,), jnp.int32),
            pltpu.VMEM((NUM_PIDS,), jnp.float32),
            pltpu.VMEM((NUM_PIDS,), jnp.float32),
            pltpu.VMEM((NUM_PIDS,), jnp.float32),
            pltpu.VMEM((NUM_PIDS,), jnp.float32),
            pltpu.VMEM((NUM_PIDS,), jnp.float32),
            pltpu.VMEM((NUM_PIDS,), jnp.float32),
        ] + [pltpu.VMEM((SEG_PER_W,), jnp.float32) for _ in range(9)],
    )(pids, dV, dJ, dT, tV, tJ, tT)


BS = 512


def _tc_body(tleaf, vc25, jm25, tp25, rd25,
             a1V, a2V, tpV, a1J, a2J, tpJ, a1T, a2T, tpT, out_ref):
    c_rk = jnp.float32(1.0 / (R_GAS * TROOM))
    c_r = jnp.float32(1.0 / R_GAS)
    rec_troom = jnp.float32(1.0 / TROOM)
    d_vj = jnp.float32(DHD_VCMAX / R_GAS)
    d_t = jnp.float32(DHD_TPU / R_GAS)

    r = 1.0 / tleaf[...]
    e_vj = jnp.exp(-d_vj * r)
    e_t = jnp.exp(-d_t * r)

    def chan(k25, a1, a2, tp, dhd, dhd_r, e):
        a1 = a1[...]
        g = a1 / (jnp.float32(dhd) - a1)
        A = a2[...] * c_rk
        B = a2[...] * c_r
        rtp = 1.0 / tp[...]
        numc = 1.0 + g * jnp.exp(dhd_r * (rtp - rec_troom))
        G = g * jnp.exp(dhd_r * rtp)
        return k25[...] * numc * jnp.exp(A - B * r) / (1.0 + G * e)

    out_ref[0] = chan(vc25, a1V, a2V, tpV, DHD_VCMAX, d_vj, e_vj)
    out_ref[1] = chan(jm25, a1J, a2J, tpJ, DHD_JMAX, d_vj, e_vj)
    out_ref[2] = chan(tp25, a1T, a2T, tpT, DHD_TPU, d_t, e_t)
    ard = jnp.float32(DHA_RD / (R_GAS * TROOM))
    brd = jnp.float32(DHA_RD / R_GAS)
    out_ref[3] = rd25[...] * jnp.exp(ard - brd * r)


def kernel(Tleaf, Vcmax25, Jmax25, TPU25, Rd25, dHa_Vcmax, dHa_Jmax, dHa_TPU,
           Topt_Vcmax, Topt_Jmax, Topt_TPU, PIDs, lengths):
    del lengths
    coefs = _sc_gather(PIDs, dHa_Vcmax, dHa_Jmax, dHa_TPU,
                       Topt_Vcmax, Topt_Jmax, Topt_TPU)
    coefs2d = [c.reshape(SEG, 1) for c in coefs]
    elems = [x.reshape(SEG, LEN) for x in (Tleaf, Vcmax25, Jmax25, TPU25, Rd25)]

    eblk = pl.BlockSpec((BS, LEN), lambda i: (i, 0))
    cblk = pl.BlockSpec((BS, 1), lambda i: (i, 0))
    out = pl.pallas_call(
        _tc_body,
        grid=(SEG // BS,),
        in_specs=[eblk] * 5 + [cblk] * 9,
        out_specs=pl.BlockSpec((4, BS, LEN), lambda i: (0, i, 0)),
        out_shape=jax.ShapeDtypeStruct((4, SEG, LEN), jnp.float32),
    )(*elems, *coefs2d)
    return out.reshape(4, TOTAL)


# speedup vs baseline: 1671.6905x; 1.3884x over previous
import functools

import jax
import jax.numpy as jnp
from jax import lax
from jax.experimental import pallas as pl
from jax.experimental.pallas import tpu as pltpu
from jax.experimental.pallas import tpu_sc as plsc

NUM_PIDS = 1024
SEG = 8192
LEN = 128
TOTAL = SEG * LEN

R_GAS = 0.0083144598
KELVIN = 273.15
TROOM = 25.0 + KELVIN
DHA_RD = 46.39
DHD_VCMAX = 200.0
DHD_JMAX = 200.0
DHD_TPU = 201.8

NC = 2
NS = 16
LANES = 16
NW = NC * NS
SEG_PER_W = SEG // NW


def _sc_gather_body(pids_hbm, dV_hbm, dJ_hbm, dT_hbm, tV_hbm, tJ_hbm, tT_hbm,
                    coef_hbm,
                    pids_v, pids8_v, dVv, dJv, dTv, tVv, tJv, tTv,
                    v_a1V, v_a2V, v_tpV, v_a1J, v_a2J, v_tpJ,
                    v_a1T, v_a2T, v_tpT, sem):
    wid = lax.axis_index("s") * NC + lax.axis_index("c")
    base = wid * SEG_PER_W
    descs = [
        pltpu.async_copy(pids_hbm.at[pl.ds(base, SEG_PER_W)], pids_v, sem),
        pltpu.async_copy(pids_hbm.at[pl.ds(0, LANES)], pids8_v, sem),
        pltpu.async_copy(dV_hbm, dVv, sem),
        pltpu.async_copy(dJ_hbm, dJv, sem),
        pltpu.async_copy(dT_hbm, dTv, sem),
        pltpu.async_copy(tV_hbm, tVv, sem),
        pltpu.async_copy(tJ_hbm, tJv, sem),
        pltpu.async_copy(tT_hbm, tTv, sem),
    ]
    for d in descs:
        d.wait()
    bufs = (v_a1V, v_a2V, v_tpV, v_a1J, v_a2J, v_tpJ, v_a1T, v_a2T, v_tpT)
    for i in range(SEG_PER_W // LANES):
        sl = pl.ds(i * LANES, LANES)
        p = pids_v[sl]
        q = plsc.load_gather(pids8_v, [jnp.right_shift(p, 7)])
        for ch, (dv, tv) in enumerate(((dVv, tVv), (dJv, tJv), (dTv, tTv))):
            bufs[3 * ch + 0][sl] = plsc.load_gather(dv, [p])
            bufs[3 * ch + 1][sl] = plsc.load_gather(dv, [q])
            bufs[3 * ch + 2][sl] = plsc.load_gather(tv, [p])
    outs = [
        pltpu.async_copy(v, coef_hbm.at[j, pl.ds(base, SEG_PER_W)], sem)
        for j, v in enumerate(bufs)
    ]
    for d in outs:
        d.wait()


def _sc_gather(pids, dV, dJ, dT, tV, tJ, tT):
    mesh = plsc.VectorSubcoreMesh(core_axis_name="c", subcore_axis_name="s",
                                  num_cores=NC, num_subcores=NS)
    return pl.kernel(
        _sc_gather_body,
        out_type=jax.ShapeDtypeStruct((9, SEG), jnp.float32),
        mesh=mesh,
        compiler_params=pltpu.CompilerParams(needs_layout_passes=False,
                                             use_tc_tiling_on_sc=False),
        scratch_types=[
            pltpu.VMEM((SEG_PER_W,), jnp.int32),
            pltpu.VMEM((LANES,), jnp.int32),
        ] + [pltpu.VMEM((NUM_PIDS,), jnp.float32) for _ in range(6)]
          + [pltpu.VMEM((SEG_PER_W,), jnp.float32) for _ in range(9)]
          + [pltpu.SemaphoreType.DMA],
    )(pids, dV, dJ, dT, tV, tJ, tT)


BS = 512


def _col(row):
    ones = jnp.ones((1, 1), dtype=jnp.float32)
    return lax.dot_general(row, ones, (((0,), (0,)), ((), ())),
                           preferred_element_type=jnp.float32)


def _tc_body(tleaf, vc25, jm25, tp25, rd25, coef, out_ref):
    c_rk = jnp.float32(1.0 / (R_GAS * TROOM))
    c_r = jnp.float32(1.0 / R_GAS)
    rec_troom = jnp.float32(1.0 / TROOM)
    d_vj = jnp.float32(DHD_VCMAX / R_GAS)
    d_t = jnp.float32(DHD_TPU / R_GAS)

    r = 1.0 / tleaf[...]
    e_vj = jnp.exp(-d_vj * r)
    e_t = jnp.exp(-d_t * r)

    def chan(k25, ch, dhd, dhd_r, e):
        a1 = _col(coef[pl.ds(3 * ch + 0, 1), :])
        a2 = _col(coef[pl.ds(3 * ch + 1, 1), :])
        tp = _col(coef[pl.ds(3 * ch + 2, 1), :])
        g = a1 / (jnp.float32(dhd) - a1)
        A = a2 * c_rk
        B = a2 * c_r
        rtp = 1.0 / tp
        numc = 1.0 + g * jnp.exp(dhd_r * (rtp - rec_troom))
        G = g * jnp.exp(dhd_r * rtp)
        return k25[...] * numc * jnp.exp(A - B * r) / (1.0 + G * e)

    out_ref[0] = chan(vc25, 0, DHD_VCMAX, d_vj, e_vj)
    out_ref[1] = chan(jm25, 1, DHD_JMAX, d_vj, e_vj)
    out_ref[2] = chan(tp25, 2, DHD_TPU, d_t, e_t)
    ard = jnp.float32(DHA_RD / (R_GAS * TROOM))
    brd = jnp.float32(DHA_RD / R_GAS)
    out_ref[3] = rd25[...] * jnp.exp(ard - brd * r)


def kernel(Tleaf, Vcmax25, Jmax25, TPU25, Rd25, dHa_Vcmax, dHa_Jmax, dHa_TPU,
           Topt_Vcmax, Topt_Jmax, Topt_TPU, PIDs, lengths):
    del lengths
    coef = _sc_gather(PIDs, dHa_Vcmax, dHa_Jmax, dHa_TPU,
                      Topt_Vcmax, Topt_Jmax, Topt_TPU)
    elems = [x.reshape(SEG, LEN) for x in (Tleaf, Vcmax25, Jmax25, TPU25, Rd25)]

    eblk = pl.BlockSpec((BS, LEN), lambda i: (i, 0))
    cblk = pl.BlockSpec((9, BS), lambda i: (0, i))
    out = pl.pallas_call(
        _tc_body,
        grid=(SEG // BS,),
        in_specs=[eblk] * 5 + [cblk],
        out_specs=pl.BlockSpec((4, BS, LEN), lambda i: (0, i, 0)),
        out_shape=jax.ShapeDtypeStruct((4, SEG, LEN), jnp.float32),
    )(*elems, coef)
    return out.reshape(4, TOTAL)
